# baseline (speedup 1.0000x reference)
import functools

import jax
import jax.numpy as jnp
from jax import lax
from jax.experimental import pallas as pl
from jax.experimental.pallas import tpu as pltpu
from jax.experimental.pallas import tpu_sc as plsc

VOCAB = 30522
H = 768
MAX_POS = 512
B, S = 256, 512
EPS = 1e-12

L = 16
NC_, NS_ = 2, 16
NW = NC_ * NS_
BPW = B // NW
T = 64
NCHUNK = S // T
HC = H // L


def _rsqrt16(v):
    iv = plsc.bitcast(v, jnp.int32)
    y = plsc.bitcast(jnp.int32(0x5F3759DF) - (iv >> 1), jnp.float32)
    half = jnp.float32(0.5) * v
    for _ in range(3):
        y = y * (jnp.float32(1.5) - half * y * y)
    return y


def _body(ids_hbm, tts_hbm, word_hbm, pos_hbm, tt_hbm, gamma_hbm, beta_hbm,
          out_hbm, pbuf, wbuf, tdel, gbuf, bbuf, ttv, ibuf, tvbuf, sem):
    wid = lax.axis_index("c") * NS_ + lax.axis_index("s")

    pltpu.sync_copy(tt_hbm, ttv)
    pltpu.sync_copy(gamma_hbm, gbuf)
    pltpu.sync_copy(beta_hbm, bbuf)
    for j in range(HC):
        ds = pl.ds(j * L, L)
        tdel[ds] = ttv[1, ds] - ttv[0, ds]

    def chunk_body(c, carry):
        s0 = c * T
        pltpu.sync_copy(pos_hbm.at[pl.ds(s0, T)], pbuf)

        def fold_row(i, cc):
            for j in range(HC):
                ds = pl.ds(j * L, L)
                pbuf[i, ds] = pbuf[i, ds] + ttv[0, ds]
            return cc
        lax.fori_loop(0, T, fold_row, 0)

        def batch_body(b, cc):
            row0 = (wid * BPW + b) * S + s0
            pltpu.sync_copy(ids_hbm.at[pl.ds(row0, T)], ibuf)
            pltpu.sync_copy(tts_hbm.at[pl.ds(row0, T)], tvbuf)
            pltpu.async_copy(word_hbm.at[ibuf], wbuf, sem).wait()

            def token_body(i, tc):
                isp = jnp.full((L,), i, dtype=jnp.int32)
                tf = plsc.load_gather(tvbuf, [isp]).astype(jnp.float32)
                sum_v = jnp.zeros((L,), jnp.float32)
                sq_v = jnp.zeros((L,), jnp.float32)
                for j in range(HC):
                    ds = pl.ds(j * L, L)
                    x = wbuf[i, ds] + pbuf[i, ds] + tdel[ds] * tf
                    wbuf[i, ds] = x
                    sum_v = sum_v + x
                    sq_v = sq_v + x * x
                s1 = jnp.sum(sum_v)
                s2 = jnp.sum(sq_v)
                mean = s1 * jnp.float32(1.0 / H)
                var = s2 * jnp.float32(1.0 / H) - mean * mean
                mean_v = jnp.full((L,), mean, dtype=jnp.float32)
                rstd_v = _rsqrt16(jnp.full((L,), var + jnp.float32(EPS),
                                           dtype=jnp.float32))
                for j in range(HC):
                    ds = pl.ds(j * L, L)
                    xn = (wbuf[i, ds] - mean_v) * rstd_v
                    wbuf[i, ds] = xn * gbuf[ds] + bbuf[ds]
                return tc
            lax.fori_loop(0, T, token_body, 0)

            pltpu.sync_copy(wbuf, out_hbm.at[pl.ds(row0, T)])
            return cc
        lax.fori_loop(0, BPW, batch_body, 0)
        return carry

    lax.fori_loop(0, NCHUNK, chunk_body, 0)


def kernel(input_ids, token_type_ids, word_embeddings, position_embeddings,
           token_type_embeddings, gamma, beta):
    ids_flat = input_ids.reshape(-1).astype(jnp.int32)
    tts_flat = token_type_ids.reshape(-1).astype(jnp.int32)

    mesh = plsc.VectorSubcoreMesh(core_axis_name="c", subcore_axis_name="s",
                                  num_cores=NC_, num_subcores=NS_)
    run = pl.kernel(
        _body,
        out_type=jax.ShapeDtypeStruct((B * S, H), jnp.float32),
        mesh=mesh,
        compiler_params=pltpu.CompilerParams(needs_layout_passes=False),
        scratch_types=[
            pltpu.VMEM((T, H), jnp.float32),
            pltpu.VMEM((T, H), jnp.float32),
            pltpu.VMEM((H,), jnp.float32),
            pltpu.VMEM((H,), jnp.float32),
            pltpu.VMEM((H,), jnp.float32),
            pltpu.VMEM((2, H), jnp.float32),
            pltpu.VMEM((T,), jnp.int32),
            pltpu.VMEM((T,), jnp.int32),
            pltpu.SemaphoreType.DMA,
        ],
    )
    out = run(ids_flat, tts_flat, word_embeddings, position_embeddings,
              token_type_embeddings, gamma, beta)
    return out.reshape(B, S, H)


# speedup vs baseline: 1.3663x; 1.3663x over previous
import functools

import jax
import jax.numpy as jnp
from jax import lax
from jax.experimental import pallas as pl
from jax.experimental.pallas import tpu as pltpu
from jax.experimental.pallas import tpu_sc as plsc

VOCAB = 30522
H = 768
MAX_POS = 512
B, S = 256, 512
EPS = 1e-12

L = 16
NC_, NS_ = 2, 16
NW = NC_ * NS_
BPW = B // NW
T = 32
NCHUNK = S // T
HC = H // L


def _rsqrt16(v):
    iv = plsc.bitcast(v, jnp.int32)
    y = plsc.bitcast(jnp.int32(0x5F3759DF) - (iv >> 1), jnp.float32)
    half = jnp.float32(0.5) * v
    for _ in range(3):
        y = y * (jnp.float32(1.5) - half * y * y)
    return y


def _body(ids_hbm, tts_hbm, word_hbm, pos_hbm, tt_hbm, gamma_hbm, beta_hbm,
          out_hbm, pbuf, wbuf2, tdel, gbuf, bbuf, ttv, ibuf, tvbuf2, sem):
    wid = lax.axis_index("c") * NS_ + lax.axis_index("s")

    pltpu.sync_copy(tt_hbm, ttv)
    pltpu.sync_copy(gamma_hbm, gbuf)
    pltpu.sync_copy(beta_hbm, bbuf)
    for j in range(HC):
        ds = pl.ds(j * L, L)
        tdel[ds] = ttv[1, ds] - ttv[0, ds]

    def chunk_body(c, carry):
        s0 = c * T
        pltpu.sync_copy(pos_hbm.at[pl.ds(s0, T)], pbuf)

        def fold_row(i, cc):
            for j in range(HC):
                ds = pl.ds(j * L, L)
                pbuf[i, ds] = pbuf[i, ds] + ttv[0, ds]
            return cc
        lax.fori_loop(0, T, fold_row, 0)

        r0 = (wid * BPW) * S + s0
        pltpu.sync_copy(ids_hbm.at[pl.ds(r0, T)], ibuf)
        pltpu.sync_copy(tts_hbm.at[pl.ds(r0, T)], tvbuf2.at[pl.ds(0, T)])
        pltpu.async_copy(word_hbm.at[ibuf], wbuf2.at[pl.ds(0, T)], sem)

        def batch_body(b, cc):
            base = (b & 1) * T
            pltpu.make_async_copy(
                word_hbm.at[ibuf], wbuf2.at[pl.ds(base, T)], sem).wait()

            @pl.when(b < BPW - 1)
            def _prefetch():
                rn = (wid * BPW + b + 1) * S + s0
                pltpu.sync_copy(ids_hbm.at[pl.ds(rn, T)], ibuf)
                pltpu.sync_copy(tts_hbm.at[pl.ds(rn, T)],
                                tvbuf2.at[pl.ds(T - base, T)])
                pltpu.async_copy(word_hbm.at[ibuf],
                                 wbuf2.at[pl.ds(T - base, T)], sem)

            def tok2(i2, tc):
                i0 = base + 2 * i2
                i1 = i0 + 1
                p0 = 2 * i2
                p1 = p0 + 1
                tf0 = plsc.load_gather(
                    tvbuf2, [jnp.full((L,), i0, jnp.int32)]
                ).astype(jnp.float32)
                tf1 = plsc.load_gather(
                    tvbuf2, [jnp.full((L,), i1, jnp.int32)]
                ).astype(jnp.float32)
                sum0 = jnp.zeros((L,), jnp.float32)
                sq0 = jnp.zeros((L,), jnp.float32)
                sum1 = jnp.zeros((L,), jnp.float32)
                sq1 = jnp.zeros((L,), jnp.float32)
                for j in range(HC):
                    ds = pl.ds(j * L, L)
                    td = tdel[ds]
                    x0 = wbuf2[i0, ds] + pbuf[p0, ds] + td * tf0
                    x1 = wbuf2[i1, ds] + pbuf[p1, ds] + td * tf1
                    wbuf2[i0, ds] = x0
                    wbuf2[i1, ds] = x1
                    sum0 = sum0 + x0
                    sq0 = sq0 + x0 * x0
                    sum1 = sum1 + x1
                    sq1 = sq1 + x1 * x1
                m0 = jnp.sum(sum0) * jnp.float32(1.0 / H)
                m1 = jnp.sum(sum1) * jnp.float32(1.0 / H)
                v0 = jnp.sum(sq0) * jnp.float32(1.0 / H) - m0 * m0
                v1 = jnp.sum(sq1) * jnp.float32(1.0 / H) - m1 * m1
                mv0 = jnp.full((L,), m0, dtype=jnp.float32)
                mv1 = jnp.full((L,), m1, dtype=jnp.float32)
                rs0 = _rsqrt16(jnp.full((L,), v0 + jnp.float32(EPS),
                                        dtype=jnp.float32))
                rs1 = _rsqrt16(jnp.full((L,), v1 + jnp.float32(EPS),
                                        dtype=jnp.float32))
                for j in range(HC):
                    ds = pl.ds(j * L, L)
                    g = gbuf[ds]
                    bb = bbuf[ds]
                    o0 = (wbuf2[i0, ds] - mv0) * rs0 * g + bb
                    o1 = (wbuf2[i1, ds] - mv1) * rs1 * g + bb
                    wbuf2[i0, ds] = o0
                    wbuf2[i1, ds] = o1
                return tc
            lax.fori_loop(0, T // 2, tok2, 0)

            row0 = (wid * BPW + b) * S + s0
            pltpu.sync_copy(wbuf2.at[pl.ds(base, T)],
                            out_hbm.at[pl.ds(row0, T)])
            return cc
        lax.fori_loop(0, BPW, batch_body, 0)
        return carry

    lax.fori_loop(0, NCHUNK, chunk_body, 0)


def kernel(input_ids, token_type_ids, word_embeddings, position_embeddings,
           token_type_embeddings, gamma, beta):
    ids_flat = input_ids.reshape(-1).astype(jnp.int32)
    tts_flat = token_type_ids.reshape(-1).astype(jnp.int32)

    mesh = plsc.VectorSubcoreMesh(core_axis_name="c", subcore_axis_name="s",
                                  num_cores=NC_, num_subcores=NS_)
    run = pl.kernel(
        _body,
        out_type=jax.ShapeDtypeStruct((B * S, H), jnp.float32),
        mesh=mesh,
        compiler_params=pltpu.CompilerParams(needs_layout_passes=False),
        scratch_types=[
            pltpu.VMEM((T, H), jnp.float32),
            pltpu.VMEM((2 * T, H), jnp.float32),
            pltpu.VMEM((H,), jnp.float32),
            pltpu.VMEM((H,), jnp.float32),
            pltpu.VMEM((H,), jnp.float32),
            pltpu.VMEM((2, H), jnp.float32),
            pltpu.VMEM((T,), jnp.int32),
            pltpu.VMEM((2 * T,), jnp.int32),
            pltpu.SemaphoreType.DMA,
        ],
    )
    out = run(ids_flat, tts_flat, word_embeddings, position_embeddings,
              token_type_embeddings, gamma, beta)
    return out.reshape(B, S, H)


# speedup vs baseline: 1.3884x; 1.0162x over previous
import functools

import jax
import jax.numpy as jnp
from jax import lax
from jax.experimental import pallas as pl
from jax.experimental.pallas import tpu as pltpu
from jax.experimental.pallas import tpu_sc as plsc

VOCAB = 30522
H = 768
MAX_POS = 512
B, S = 256, 512
EPS = 1e-12

L = 16
NC_, NS_ = 2, 16
NW = NC_ * NS_
BPW = B // NW
T = 32
NCHUNK = S // T
HC = H // L


def _rsqrt16(v):
    iv = plsc.bitcast(v, jnp.int32)
    y = plsc.bitcast(jnp.int32(0x5F3759DF) - (iv >> 1), jnp.float32)
    half = jnp.float32(0.5) * v
    for _ in range(3):
        y = y * (jnp.float32(1.5) - half * y * y)
    return y


def _body(ids_hbm, tts_hbm, word_hbm, pos_hbm, tt_hbm, gamma_hbm, beta_hbm,
          out_hbm, pbuf2, wbuf2, gbuf, bbuf, ttv, ibuf, tvbuf2, sem):
    wid = lax.axis_index("c") * NS_ + lax.axis_index("s")
    iota = lax.iota(jnp.int32, L)

    pltpu.sync_copy(tt_hbm, ttv)
    pltpu.sync_copy(gamma_hbm, gbuf)
    pltpu.sync_copy(beta_hbm, bbuf)

    def chunk_body(c, carry):
        s0 = c * T
        pltpu.sync_copy(pos_hbm.at[pl.ds(s0, T)], pbuf2.at[pl.ds(0, T)])

        def fold_row(i, cc):
            for j in range(HC):
                ds = pl.ds(j * L, L)
                p = pbuf2[i, ds]
                pbuf2[T + i, ds] = p + ttv[1, ds]
                pbuf2[i, ds] = p + ttv[0, ds]
            return cc
        lax.fori_loop(0, T, fold_row, 0)

        r0 = (wid * BPW) * S + s0
        pltpu.sync_copy(ids_hbm.at[pl.ds(r0, T)], ibuf)
        pltpu.sync_copy(tts_hbm.at[pl.ds(r0, T)], tvbuf2.at[pl.ds(0, T)])
        pltpu.async_copy(word_hbm.at[ibuf], wbuf2.at[pl.ds(0, T)], sem)

        def batch_body(b, cc):
            base = (b & 1) * T
            pltpu.make_async_copy(
                word_hbm.at[ibuf], wbuf2.at[pl.ds(base, T)], sem).wait()

            @pl.when(b < BPW - 1)
            def _prefetch():
                rn = (wid * BPW + b + 1) * S + s0
                pltpu.sync_copy(ids_hbm.at[pl.ds(rn, T)], ibuf)
                pltpu.sync_copy(tts_hbm.at[pl.ds(rn, T)],
                                tvbuf2.at[pl.ds(T - base, T)])
                pltpu.async_copy(word_hbm.at[ibuf],
                                 wbuf2.at[pl.ds(T - base, T)], sem)

            def tok2(i2, tc):
                i0 = base + 2 * i2
                i1 = i0 + 1
                p0 = 2 * i2
                p1 = p0 + 1
                trow0 = plsc.load_gather(
                    tvbuf2, [jnp.full((L,), i0, jnp.int32)]
                ) * T + jnp.full((L,), p0, jnp.int32)
                trow1 = plsc.load_gather(
                    tvbuf2, [jnp.full((L,), i1, jnp.int32)]
                ) * T + jnp.full((L,), p1, jnp.int32)
                sum0 = jnp.zeros((L,), jnp.float32)
                sq0 = jnp.zeros((L,), jnp.float32)
                sum1 = jnp.zeros((L,), jnp.float32)
                sq1 = jnp.zeros((L,), jnp.float32)
                for j in range(HC):
                    ds = pl.ds(j * L, L)
                    col = iota + (j * L)
                    pc0 = plsc.load_gather(pbuf2, [trow0, col])
                    pc1 = plsc.load_gather(pbuf2, [trow1, col])
                    x0 = wbuf2[i0, ds] + pc0
                    x1 = wbuf2[i1, ds] + pc1
                    wbuf2[i0, ds] = x0
                    wbuf2[i1, ds] = x1
                    sum0 = sum0 + x0
                    sq0 = sq0 + x0 * x0
                    sum1 = sum1 + x1
                    sq1 = sq1 + x1 * x1
                m0 = jnp.sum(sum0) * jnp.float32(1.0 / H)
                m1 = jnp.sum(sum1) * jnp.float32(1.0 / H)
                v0 = jnp.sum(sq0) * jnp.float32(1.0 / H) - m0 * m0
                v1 = jnp.sum(sq1) * jnp.float32(1.0 / H) - m1 * m1
                mv0 = jnp.full((L,), m0, dtype=jnp.float32)
                mv1 = jnp.full((L,), m1, dtype=jnp.float32)
                rs0 = _rsqrt16(jnp.full((L,), v0 + jnp.float32(EPS),
                                        dtype=jnp.float32))
                rs1 = _rsqrt16(jnp.full((L,), v1 + jnp.float32(EPS),
                                        dtype=jnp.float32))
                for j in range(HC):
                    ds = pl.ds(j * L, L)
                    g = gbuf[ds]
                    bb = bbuf[ds]
                    o0 = (wbuf2[i0, ds] - mv0) * rs0 * g + bb
                    o1 = (wbuf2[i1, ds] - mv1) * rs1 * g + bb
                    wbuf2[i0, ds] = o0
                    wbuf2[i1, ds] = o1
                return tc
            lax.fori_loop(0, T // 2, tok2, 0)

            row0 = (wid * BPW + b) * S + s0
            pltpu.sync_copy(wbuf2.at[pl.ds(base, T)],
                            out_hbm.at[pl.ds(row0, T)])
            return cc
        lax.fori_loop(0, BPW, batch_body, 0)
        return carry

    lax.fori_loop(0, NCHUNK, chunk_body, 0)


def kernel(input_ids, token_type_ids, word_embeddings, position_embeddings,
           token_type_embeddings, gamma, beta):
    ids_flat = input_ids.reshape(-1).astype(jnp.int32)
    tts_flat = token_type_ids.reshape(-1).astype(jnp.int32)

    mesh = plsc.VectorSubcoreMesh(core_axis_name="c", subcore_axis_name="s",
                                  num_cores=NC_, num_subcores=NS_)
    run = pl.kernel(
        _body,
        out_type=jax.ShapeDtypeStruct((B * S, H), jnp.float32),
        mesh=mesh,
        compiler_params=pltpu.CompilerParams(needs_layout_passes=False),
        scratch_types=[
            pltpu.VMEM((2 * T, H), jnp.float32),
            pltpu.VMEM((2 * T, H), jnp.float32),
            pltpu.VMEM((H,), jnp.float32),
            pltpu.VMEM((H,), jnp.float32),
            pltpu.VMEM((2, H), jnp.float32),
            pltpu.VMEM((T,), jnp.int32),
            pltpu.VMEM((2 * T,), jnp.int32),
            pltpu.SemaphoreType.DMA,
        ],
    )
    out = run(ids_flat, tts_flat, word_embeddings, position_embeddings,
              token_type_embeddings, gamma, beta)
    return out.reshape(B, S, H)
